# Initial kernel scaffold; baseline (speedup 1.0000x reference)
import jax
import jax.numpy as jnp
from jax.experimental import pallas as pl


def kernel(x, W_feat, W_cls, filter_K):
    raise NotImplementedError("write your pallas kernel here")


# speedup vs baseline: 1.1673x; 1.1673x over previous
import functools

import jax
import jax.numpy as jnp
from jax.experimental import pallas as pl

B, D_IN, D, C = 16384, 128, 64, 1000
N = C + B
N_PAD = 17408
BLK_A = 512
BLK_C = 1088


def _row_stats(p):
    m = jnp.max(p, axis=1, keepdims=True)
    e = jnp.exp(p - m)
    s = jnp.sum(e, axis=1, keepdims=True)
    lse = jnp.log(s) + m
    sp = jnp.sum(e * p, axis=1, keepdims=True) / s
    ent = jnp.maximum((lse - sp)[:, 0], 0.0)
    col = jax.lax.broadcasted_iota(jnp.int32, p.shape, 1)
    yid = jnp.min(jnp.where(p == m, col, p.shape[1]), axis=1)
    return ent, yid


def _feat_body(x_ref, wf_ref, wc_ref, z_ref, zn_ref, ent_ref, yid_ref):
    x = x_ref[...]
    z = jnp.dot(x, wf_ref[...], preferred_element_type=jnp.float32)
    p = jnp.dot(z, wc_ref[...], preferred_element_type=jnp.float32)
    ent, yid = _row_stats(p)
    z_ref[...] = z
    n = jnp.sqrt(jnp.sum(z * z, axis=1, keepdims=True))
    zn_ref[...] = z / jnp.maximum(n, 1e-12)
    ent_ref[...] = ent.reshape(1, 1, BLK_A)
    yid_ref[...] = yid.reshape(1, 1, BLK_A)


def _warm_body(wcb_ref, wc_ref, went_ref, wyid_ref, wsup_ref):
    wcb = wcb_ref[...]
    wp = jax.lax.dot_general(wcb, wc_ref[...], (((0,), (0,)), ((), ())),
                             preferred_element_type=jnp.float32)
    ent, yid = _row_stats(wp)
    went_ref[...] = ent.reshape(1, 1, 256)
    wyid_ref[...] = yid.reshape(1, 1, 256)
    n = jnp.sqrt(jnp.sum(wcb * wcb, axis=0, keepdims=True))
    wsup_ref[...] = wcb / jnp.maximum(n, 1e-12)


def _wsum_body(zn_ref, y_ref, m_ref, w_ref):
    i = pl.program_id(0)
    zn = zn_ref[...]
    y = y_ref[0, 0, :]
    msk = m_ref[0, 0, :]
    col = jax.lax.broadcasted_iota(jnp.int32, (BLK_C, C), 1)
    oh = jnp.where(col == y[:, None], msk[:, None], 0.0)
    acc = jax.lax.dot_general(zn, oh, (((0,), (0,)), ((), ())),
                              preferred_element_type=jnp.float32)

    @pl.when(i == 0)
    def _():
        w_ref[...] = acc

    @pl.when(i > 0)
    def _():
        w_ref[...] += acc


def _out_body(z_ref, w_ref, o_ref):
    w = w_ref[...]
    n = jnp.sqrt(jnp.sum(w * w, axis=0, keepdims=True))
    wn = w / jnp.maximum(n, 1e-12)
    o_ref[...] = jnp.dot(z_ref[...], wn, preferred_element_type=jnp.float32)


def _feat_call(x, W_feat, W_cls):
    grid = B // BLK_A
    return pl.pallas_call(
        _feat_body,
        grid=(grid,),
        in_specs=[
            pl.BlockSpec((BLK_A, D_IN), lambda i: (i, 0)),
            pl.BlockSpec((D_IN, D), lambda i: (0, 0)),
            pl.BlockSpec((D, C), lambda i: (0, 0)),
        ],
        out_specs=[
            pl.BlockSpec((BLK_A, D), lambda i: (i, 0)),
            pl.BlockSpec((BLK_A, D), lambda i: (i, 0)),
            pl.BlockSpec((1, 1, BLK_A), lambda i: (i, 0, 0)),
            pl.BlockSpec((1, 1, BLK_A), lambda i: (i, 0, 0)),
        ],
        out_shape=[
            jax.ShapeDtypeStruct((B, D), jnp.float32),
            jax.ShapeDtypeStruct((B, D), jnp.float32),
            jax.ShapeDtypeStruct((grid, 1, BLK_A), jnp.float32),
            jax.ShapeDtypeStruct((grid, 1, BLK_A), jnp.int32),
        ],
    )(x, W_feat, W_cls)


def _warm_call(W_cls):
    wc_pad = jnp.pad(W_cls, ((0, 0), (0, 1024 - C)))
    went, wyid, wsup = pl.pallas_call(
        _warm_body,
        grid=(4,),
        in_specs=[
            pl.BlockSpec((D, 256), lambda i: (0, i)),
            pl.BlockSpec((D, C), lambda i: (0, 0)),
        ],
        out_specs=[
            pl.BlockSpec((1, 1, 256), lambda i: (i, 0, 0)),
            pl.BlockSpec((1, 1, 256), lambda i: (i, 0, 0)),
            pl.BlockSpec((D, 256), lambda i: (0, i)),
        ],
        out_shape=[
            jax.ShapeDtypeStruct((4, 1, 256), jnp.float32),
            jax.ShapeDtypeStruct((4, 1, 256), jnp.int32),
            jax.ShapeDtypeStruct((D, 1024), jnp.float32),
        ],
    )(wc_pad, W_cls)
    return (went.reshape(-1)[:C], wyid.reshape(-1)[:C],
            wsup[:, :C].T)


def _wsum_call(zn_all, y_all, mask):
    grid = N_PAD // BLK_C
    return pl.pallas_call(
        _wsum_body,
        grid=(grid,),
        in_specs=[
            pl.BlockSpec((BLK_C, D), lambda i: (i, 0)),
            pl.BlockSpec((1, 1, BLK_C), lambda i: (i, 0, 0)),
            pl.BlockSpec((1, 1, BLK_C), lambda i: (i, 0, 0)),
        ],
        out_specs=pl.BlockSpec((D, C), lambda i: (0, 0)),
        out_shape=jax.ShapeDtypeStruct((D, C), jnp.float32),
    )(zn_all, y_all.reshape(grid, 1, BLK_C), mask.reshape(grid, 1, BLK_C))


def _out_call(z, W):
    return pl.pallas_call(
        _out_body,
        grid=(B // BLK_A,),
        in_specs=[
            pl.BlockSpec((BLK_A, D), lambda i: (i, 0)),
            pl.BlockSpec((D, C), lambda i: (0, 0)),
        ],
        out_specs=pl.BlockSpec((BLK_A, C), lambda i: (i, 0)),
        out_shape=jax.ShapeDtypeStruct((B, C), jnp.float32),
    )(z, W)


def _select_mask(y_all, ent_all, filter_K):
    order = jnp.lexsort((ent_all, y_all))
    sorted_y = y_all[order]
    pos = jnp.arange(N, dtype=jnp.int32)
    start = jnp.full((C,), N, dtype=jnp.int32).at[sorted_y].min(pos)
    rank_sorted = pos - start[sorted_y]
    rank = jnp.zeros((N,), dtype=jnp.int32).at[order].set(rank_sorted)
    return (rank < filter_K).astype(jnp.float32)


def kernel(x, W_feat, W_cls, filter_K):
    z, zn, ent3, yid3 = _feat_call(x, W_feat, W_cls)
    went, wyid, wsup_n = _warm_call(W_cls)
    ent_all = jnp.concatenate([went, ent3.reshape(-1)])
    y_all = jnp.concatenate([wyid, yid3.reshape(-1)])
    mask = _select_mask(y_all, ent_all, filter_K)
    zn_all = jnp.concatenate([wsup_n, zn], axis=0)
    pad = N_PAD - N
    zn_all = jnp.pad(zn_all, ((0, pad), (0, 0)))
    y_pad = jnp.pad(y_all, (0, pad))
    m_pad = jnp.pad(mask, (0, pad))
    W = _wsum_call(zn_all, y_pad, m_pad)
    return _out_call(z, W)


# speedup vs baseline: 2.0606x; 1.7653x over previous
import functools

import jax
import jax.numpy as jnp
from jax import lax
from jax.experimental import pallas as pl
from jax.experimental.pallas import tpu as pltpu
from jax.experimental.pallas import tpu_sc as plsc

B, D_IN, D, C = 16384, 128, 64, 1000
N = C + B
N_PAD = 17408
BLK_A = 512
BLK_C = 1088


def _row_stats(p):
    m = jnp.max(p, axis=1, keepdims=True)
    e = jnp.exp(p - m)
    s = jnp.sum(e, axis=1, keepdims=True)
    lse = jnp.log(s) + m
    sp = jnp.sum(e * p, axis=1, keepdims=True) / s
    ent = jnp.maximum((lse - sp)[:, 0], 0.0)
    col = jax.lax.broadcasted_iota(jnp.int32, p.shape, 1)
    yid = jnp.min(jnp.where(p == m, col, p.shape[1]), axis=1)
    return ent, yid


def _feat_body(x_ref, wf_ref, wc_ref, z_ref, zn_ref, ent_ref, yid_ref):
    x = x_ref[...]
    z = jnp.dot(x, wf_ref[...], preferred_element_type=jnp.float32)
    p = jnp.dot(z, wc_ref[...], preferred_element_type=jnp.float32)
    ent, yid = _row_stats(p)
    z_ref[...] = z
    n = jnp.sqrt(jnp.sum(z * z, axis=1, keepdims=True))
    zn_ref[...] = z / jnp.maximum(n, 1e-12)
    ent_ref[...] = ent.reshape(1, 1, BLK_A)
    yid_ref[...] = yid.reshape(1, 1, BLK_A)


def _warm_body(wcb_ref, wc_ref, went_ref, wyid_ref, wsup_ref):
    wcb = wcb_ref[...]
    wp = jax.lax.dot_general(wcb, wc_ref[...], (((0,), (0,)), ((), ())),
                             preferred_element_type=jnp.float32)
    ent, yid = _row_stats(wp)
    went_ref[...] = ent.reshape(1, 1, 256)
    wyid_ref[...] = yid.reshape(1, 1, 256)
    n = jnp.sqrt(jnp.sum(wcb * wcb, axis=0, keepdims=True))
    wsup_ref[...] = wcb / jnp.maximum(n, 1e-12)


def _wsum_body(zn_ref, y_ref, m_ref, w_ref):
    i = pl.program_id(0)
    zn = zn_ref[...]
    y = y_ref[0, 0, :]
    msk = m_ref[0, 0, :]
    col = jax.lax.broadcasted_iota(jnp.int32, (BLK_C, C), 1)
    oh = jnp.where(col == y[:, None], msk[:, None], 0.0)
    acc = jax.lax.dot_general(zn, oh, (((0,), (0,)), ((), ())),
                              preferred_element_type=jnp.float32)

    @pl.when(i == 0)
    def _():
        w_ref[...] = acc

    @pl.when(i > 0)
    def _():
        w_ref[...] += acc


def _out_body(z_ref, w_ref, o_ref):
    w = w_ref[...]
    n = jnp.sqrt(jnp.sum(w * w, axis=0, keepdims=True))
    wn = w / jnp.maximum(n, 1e-12)
    o_ref[...] = jnp.dot(z_ref[...], wn, preferred_element_type=jnp.float32)


def _feat_call(x, W_feat, W_cls):
    grid = B // BLK_A
    return pl.pallas_call(
        _feat_body,
        grid=(grid,),
        in_specs=[
            pl.BlockSpec((BLK_A, D_IN), lambda i: (i, 0)),
            pl.BlockSpec((D_IN, D), lambda i: (0, 0)),
            pl.BlockSpec((D, C), lambda i: (0, 0)),
        ],
        out_specs=[
            pl.BlockSpec((BLK_A, D), lambda i: (i, 0)),
            pl.BlockSpec((BLK_A, D), lambda i: (i, 0)),
            pl.BlockSpec((1, 1, BLK_A), lambda i: (i, 0, 0)),
            pl.BlockSpec((1, 1, BLK_A), lambda i: (i, 0, 0)),
        ],
        out_shape=[
            jax.ShapeDtypeStruct((B, D), jnp.float32),
            jax.ShapeDtypeStruct((B, D), jnp.float32),
            jax.ShapeDtypeStruct((grid, 1, BLK_A), jnp.float32),
            jax.ShapeDtypeStruct((grid, 1, BLK_A), jnp.int32),
        ],
    )(x, W_feat, W_cls)


def _warm_call(W_cls):
    wc_pad = jnp.pad(W_cls, ((0, 0), (0, 1024 - C)))
    went, wyid, wsup = pl.pallas_call(
        _warm_body,
        grid=(4,),
        in_specs=[
            pl.BlockSpec((D, 256), lambda i: (0, i)),
            pl.BlockSpec((D, C), lambda i: (0, 0)),
        ],
        out_specs=[
            pl.BlockSpec((1, 1, 256), lambda i: (i, 0, 0)),
            pl.BlockSpec((1, 1, 256), lambda i: (i, 0, 0)),
            pl.BlockSpec((D, 256), lambda i: (0, i)),
        ],
        out_shape=[
            jax.ShapeDtypeStruct((4, 1, 256), jnp.float32),
            jax.ShapeDtypeStruct((4, 1, 256), jnp.int32),
            jax.ShapeDtypeStruct((D, 1024), jnp.float32),
        ],
    )(wc_pad, W_cls)
    return (went.reshape(-1)[:C], wyid.reshape(-1)[:C],
            wsup[:, :C].T)


def _wsum_call(zn_all, y_all, mask):
    grid = N_PAD // BLK_C
    return pl.pallas_call(
        _wsum_body,
        grid=(grid,),
        in_specs=[
            pl.BlockSpec((BLK_C, D), lambda i: (i, 0)),
            pl.BlockSpec((1, 1, BLK_C), lambda i: (i, 0, 0)),
            pl.BlockSpec((1, 1, BLK_C), lambda i: (i, 0, 0)),
        ],
        out_specs=pl.BlockSpec((D, C), lambda i: (0, 0)),
        out_shape=jax.ShapeDtypeStruct((D, C), jnp.float32),
    )(zn_all, y_all.reshape(grid, 1, BLK_C), mask.reshape(grid, 1, BLK_C))


def _out_call(z, W):
    return pl.pallas_call(
        _out_body,
        grid=(B // BLK_A,),
        in_specs=[
            pl.BlockSpec((BLK_A, D), lambda i: (i, 0)),
            pl.BlockSpec((D, C), lambda i: (0, 0)),
        ],
        out_specs=pl.BlockSpec((BLK_A, C), lambda i: (i, 0)),
        out_shape=jax.ShapeDtypeStruct((B, C), jnp.float32),
    )(z, W)


L = 16
CH = N_PAD // 16
GRP = CH // L
CPAD = 1024
S_CAP = 512
SP = S_CAP // 16
HB = S_CAP * L
MB = SP * L


def _sc_mask_body(y_hbm, eb_hbm, kv_hbm, mask_hbm,
                  yv, ev, actb, maskv, cntv, slotl, cbuf, totv,
                  totm, hist1, comb2, comb1, tel, til, remv, kvv, skb, csb,
                  sh_cnth, sh_tot, sh_slot, sh_hist, sh_te, sh_ti):
    sid = lax.axis_index("s")
    cid = lax.axis_index("c")
    base = sid * CH
    lane = lax.iota(jnp.int32, L)
    zero16 = jnp.zeros((L,), jnp.int32)
    ones16 = jnp.ones((L,), jnp.int32)
    tmask = jnp.full((L,), True)

    pltpu.sync_copy(y_hbm.at[pl.ds(base, CH)], yv)
    pltpu.sync_copy(eb_hbm.at[pl.ds(base, CH)], ev)
    pltpu.sync_copy(kv_hbm, kvv)
    kk = kvv[...][0]

    def _zero(ref, nwords):
        def zb(i, _):
            ref[pl.ds(i * L, L)] = zero16
            return 0
        lax.fori_loop(0, nwords // L, zb, 0)

    _zero(cntv, CPAD)
    _zero(actb, CH + L)

    def dup_add(hist_ref, tsize, idx, val, m):
        k = jnp.where(m, idx, tsize - 1)
        v = jnp.where(m, val, 0)
        sk, sv = plsc.sort_key_val(k, v)
        skb[pl.ds(8, L)] = sk
        prev = skb[pl.ds(7, L)]
        nxt = skb[pl.ds(9, L)]
        is_start = (lane == 0) | (sk != prev)
        is_last = (lane == L - 1) | (sk != nxt)
        rstart = plsc.cummax(jnp.where(is_start, lane, 0))
        cs = jnp.cumsum(sv)
        csb[pl.ds(0, L)] = zero16
        csb[pl.ds(1, L)] = cs
        excl = plsc.load_gather(csb, [rstart])
        plsc.addupdate_scatter(hist_ref, [sk], cs - excl, mask=is_last)

    def h1(g, _):
        y16 = yv[pl.ds(g * L, L)]
        dup_add(cntv, CPAD, y16, ones16, tmask)
        return 0
    lax.fori_loop(0, GRP, h1, 0)

    pltpu.sync_copy(cntv, sh_cnth.at[sid])
    plsc.subcore_barrier()

    coff = sid * (CPAD // 16)

    def cp1(t, _):
        pltpu.sync_copy(sh_cnth.at[t, pl.ds(coff, CPAD // 16)], cbuf.at[t])
        return 0
    lax.fori_loop(0, 16, cp1, 0)

    acc4 = []
    for j in range(4):
        def rd(t, a, j=j):
            return a + cbuf[t, pl.ds(j * L, L)]
        acc4.append(lax.fori_loop(0, 16, rd, zero16))

    flags, slotrk = [], []
    carry = jnp.int32(0)
    for j in range(4):
        f = (acc4[j] > kk).astype(jnp.int32)
        cs = jnp.cumsum(f)
        slotrk.append(cs - f + carry)
        flags.append(f)
        carry = carry + jnp.sum(f)
    mytot = carry

    totv[...] = jnp.broadcast_to(mytot, (L,)).astype(jnp.int32)
    pltpu.sync_copy(totv, sh_tot.at[sid])
    plsc.subcore_barrier()
    pltpu.sync_copy(sh_tot, totm)

    def pf(t, c):
        off, tot = c
        v = totm[t, pl.ds(0, L)][0]
        return (off + jnp.where(t < sid, v, 0), tot + v)
    myoff, s_total = lax.fori_loop(0, 16, pf, (jnp.int32(0), jnp.int32(0)))

    for j in range(4):
        sv = jnp.where(flags[j] > 0, slotrk[j] + myoff, -1)
        slotl[pl.ds(coff + j * L, L)] = sv
    pltpu.sync_copy(slotl.at[pl.ds(coff, CPAD // 16)],
                    sh_slot.at[pl.ds(coff, CPAD // 16)])

    soff = sid * SP
    km1 = jnp.broadcast_to(kk - 1, (L,)).astype(jnp.int32)
    for j in range(SP // L):
        remv[pl.ds(soff + j * L, L)] = km1
        tel[pl.ds(soff + j * L, L)] = zero16
        til[pl.ds(soff + j * L, L)] = zero16
    pltpu.sync_copy(tel.at[pl.ds(soff, SP)], sh_te.at[pl.ds(soff, SP)])
    pltpu.sync_copy(til.at[pl.ds(soff, SP)], sh_ti.at[pl.ds(soff, SP)])
    plsc.subcore_barrier()
    pltpu.sync_copy(sh_slot, slotl)
    pltpu.sync_copy(sh_te, tel)
    pltpu.sync_copy(sh_ti, til)

    def cw(g, na):
        y16 = yv[pl.ds(g * L, L)]
        s16 = plsc.load_gather(slotl, [y16])
        act = s16 >= 0
        lidx = lane + g * L
        plsc.store_compressed(actb.at[pl.ds(na, L)], lidx, mask=act)
        return na + jnp.sum(act.astype(jnp.int32))
    na = lax.fori_loop(0, GRP, cw, jnp.int32(0))

    @pl.when(s_total > 0)
    def _rounds():
        def rbody(r, _):
            _zero(hist1, HB)
            shv = jnp.where(r < 8, 28 - 4 * r, 44 - 4 * r)
            es4 = jnp.clip(32 - 4 * r, 0, 31)
            is4 = jnp.clip(48 - 4 * r, 0, 31)
            ngr = (na + L - 1) // L

            def walk(g, _):
                lidx = actb[pl.ds(g * L, L)]
                lm = (lane + g * L) < na
                y16 = plsc.load_gather(yv, [lidx], mask=lm)
                e16 = plsc.load_gather(ev, [lidx], mask=lm)
                s16 = plsc.load_gather(slotl, [y16], mask=lm)
                gi = lidx + base
                te16 = plsc.load_gather(tel, [s16], mask=lm)
                ti16 = plsc.load_gather(til, [s16], mask=lm)
                eq = (((e16 >> es4) == (te16 >> es4))
                      & ((gi >> is4) == (ti16 >> is4)))
                act = lm & eq
                src = jnp.where(r < 8, e16, gi)
                dig = (src >> shv) & 15
                dup_add(hist1, HB, s16 * L + dig, ones16, act)
                return 0
            lax.fori_loop(0, ngr, walk, 0)

            pltpu.sync_copy(hist1, sh_hist.at[sid])
            plsc.subcore_barrier()

            boff = sid * MB

            def cp2(t, _):
                pltpu.sync_copy(sh_hist.at[t, pl.ds(boff, MB)], comb2.at[t])
                return 0
            lax.fori_loop(0, 16, cp2, 0)

            def rs(i, _):
                def inr(t, a):
                    return a + comb2[t, pl.ds(i * L, L)]
                comb1[pl.ds(i * L, L)] = lax.fori_loop(0, 16, inr, zero16)
                return 0
            lax.fori_loop(0, MB // L, rs, 0)

            for jg in range(SP // L):
                remvec = remv[pl.ds(soff + jg * L, L)]
                tevec = tel[pl.ds(soff + jg * L, L)]
                tivec = til[pl.ds(soff + jg * L, L)]
                nrem, nte, nti = zero16, zero16, zero16
                for jl in range(L):
                    j = jg * L + jl
                    binv = comb1[pl.ds(j * L, L)]
                    cums = jnp.cumsum(binv)
                    excl = cums - binv
                    remj = remvec[jl]
                    le = excl <= remj
                    nd = jnp.sum(le.astype(jnp.int32)) - 1
                    exclsel = jnp.sum(jnp.where(lane == nd, excl, 0))
                    add = nd << shv
                    tej = tevec[jl] + jnp.where(r < 8, add, 0)
                    tij = tivec[jl] + jnp.where(r < 8, 0, add)
                    nrem = jnp.where(lane == jl, remj - exclsel, nrem)
                    nte = jnp.where(lane == jl, tej, nte)
                    nti = jnp.where(lane == jl, tij, nti)
                remv[pl.ds(soff + jg * L, L)] = nrem
                tel[pl.ds(soff + jg * L, L)] = nte
                til[pl.ds(soff + jg * L, L)] = nti
            pltpu.sync_copy(tel.at[pl.ds(soff, SP)], sh_te.at[pl.ds(soff, SP)])
            pltpu.sync_copy(til.at[pl.ds(soff, SP)], sh_ti.at[pl.ds(soff, SP)])
            plsc.subcore_barrier()
            pltpu.sync_copy(sh_te, tel)
            pltpu.sync_copy(sh_ti, til)
            return 0
        lax.fori_loop(0, 12, rbody, 0)

    def fm(g, _):
        y16 = yv[pl.ds(g * L, L)]
        e16 = ev[pl.ds(g * L, L)]
        gi = lane + (g * L + base)
        s16 = plsc.load_gather(slotl, [y16])
        val = s16 >= 0
        te16 = plsc.load_gather(tel, [s16], mask=val)
        ti16 = plsc.load_gather(til, [s16], mask=val)
        sel = (e16 < te16) | ((e16 == te16) & (gi <= ti16))
        maskv[pl.ds(g * L, L)] = jnp.where(val & jnp.logical_not(sel), 0.0, 1.0)
        return 0
    lax.fori_loop(0, GRP, fm, 0)

    @pl.when(cid == 0)
    def _():
        pltpu.sync_copy(maskv, mask_hbm.at[pl.ds(base, CH)])


def _sc_mask_call(y_pad, eb, kvec):
    mesh = plsc.VectorSubcoreMesh(core_axis_name="c", subcore_axis_name="s")
    shared = [
        pltpu.VMEM_SHARED((16, CPAD), jnp.int32),
        pltpu.VMEM_SHARED((16, L), jnp.int32),
        pltpu.VMEM_SHARED((CPAD,), jnp.int32),
        pltpu.VMEM_SHARED((16, HB), jnp.int32),
        pltpu.VMEM_SHARED((S_CAP,), jnp.int32),
        pltpu.VMEM_SHARED((S_CAP,), jnp.int32),
    ]
    local = [
        pltpu.VMEM((CH,), jnp.int32),
        pltpu.VMEM((CH,), jnp.int32),
        pltpu.VMEM((CH + L,), jnp.int32),
        pltpu.VMEM((CH,), jnp.float32),
        pltpu.VMEM((CPAD,), jnp.int32),
        pltpu.VMEM((CPAD,), jnp.int32),
        pltpu.VMEM((16, CPAD // 16), jnp.int32),
        pltpu.VMEM((L,), jnp.int32),
        pltpu.VMEM((16, L), jnp.int32),
        pltpu.VMEM((HB,), jnp.int32),
        pltpu.VMEM((16, MB), jnp.int32),
        pltpu.VMEM((MB,), jnp.int32),
        pltpu.VMEM((S_CAP,), jnp.int32),
        pltpu.VMEM((S_CAP,), jnp.int32),
        pltpu.VMEM((S_CAP,), jnp.int32),
        pltpu.VMEM((L,), jnp.int32),
        pltpu.VMEM((40,), jnp.int32),
        pltpu.VMEM((32,), jnp.int32),
    ]
    kcall = pl.kernel(
        _sc_mask_body,
        mesh=mesh,
        out_type=jax.ShapeDtypeStruct((N_PAD,), jnp.float32),
        scratch_types=local + shared,
        compiler_params=pltpu.CompilerParams(needs_layout_passes=False),
    )
    return kcall(y_pad, eb, kvec)


def kernel(x, W_feat, W_cls, filter_K):
    z, zn, ent3, yid3 = _feat_call(x, W_feat, W_cls)
    went, wyid, wsup_n = _warm_call(W_cls)
    pad = N_PAD - N
    ent_all = jnp.concatenate([went, ent3.reshape(-1), jnp.zeros((pad,))])
    y_pad = jnp.concatenate(
        [wyid, yid3.reshape(-1),
         jnp.full((pad,), C, jnp.int32)])
    eb = jax.lax.bitcast_convert_type(ent_all.astype(jnp.float32), jnp.int32)
    kvec = jnp.full((16,), filter_K, jnp.int32)
    m_pad = _sc_mask_call(y_pad, eb, kvec)
    zn_all = jnp.concatenate([wsup_n, zn], axis=0)
    zn_all = jnp.pad(zn_all, ((0, pad), (0, 0)))
    W = _wsum_call(zn_all, y_pad, m_pad)
    return _out_call(z, W)
